# Initial kernel scaffold; baseline (speedup 1.0000x reference)
import jax
import jax.numpy as jnp
from jax.experimental import pallas as pl


def kernel(x, pos, edge_index, batch, W_emb_in, b_emb_in, We1, be1, We2, be2, Wc1, bc1, Wc2, Wn1, bn1, Wn2, bn2, W_emb_out, b_emb_out, Wg1, bg1, gamma, beta, Wg2, bg2, Wf1, bf1, Wf2, bf2):
    raise NotImplementedError("write your pallas kernel here")


# speedup vs baseline: 1.3075x; 1.3075x over previous
import functools

import jax
import jax.numpy as jnp
from jax import lax
from jax.experimental import pallas as pl
from jax.experimental.pallas import tpu as pltpu
from jax.experimental.pallas import tpu_sc as plsc

H = 16


def _edge_mlp_body(rrow_ref, rcol_ref, vecs_ref, we2_ref, m_ref):
    rrow = rrow_ref[...]
    rcol = rcol_ref[...]
    vecs = vecs_ref[...]
    aA = vecs[0]
    aB = vecs[1]
    aR = vecs[2]
    c0 = vecs[3]
    be2 = vecs[4]
    xr = rrow[:, 0:1]
    xc = rcol[:, 0:1]
    d = rrow[:, 1:4] - rcol[:, 1:4]
    radial = jnp.sum(d * d, axis=1, keepdims=True)
    pre = xr * aA[None, :] + xc * aB[None, :] + radial * aR[None, :] + c0[None, :]
    u = pre * jax.nn.sigmoid(pre)
    v = jnp.dot(u, we2_ref[...], preferred_element_type=jnp.float32) + be2[None, :]
    m_ref[...] = v * jax.nn.sigmoid(v)


def _edge_mlp(rrow, rcol, vecs, we2, eblk=8192):
    e_pad = rrow.shape[0]
    grid = (e_pad // eblk,)
    return pl.pallas_call(
        _edge_mlp_body,
        grid=grid,
        in_specs=[
            pl.BlockSpec((eblk, 8), lambda i: (i, 0)),
            pl.BlockSpec((eblk, 8), lambda i: (i, 0)),
            pl.BlockSpec((8, H), lambda i: (0, 0)),
            pl.BlockSpec((H, H), lambda i: (0, 0)),
        ],
        out_specs=pl.BlockSpec((eblk, H), lambda i: (i, 0)),
        out_shape=jax.ShapeDtypeStruct((e_pad, H), jnp.float32),
    )(rrow, rcol, vecs, we2)


def _node_body(x_ref, agg0_ref, agg1_ref, vecs_ref, mats_ref, h3_ref):
    x = x_ref[...]
    agg = agg0_ref[...] + agg1_ref[...]
    vecs = vecs_ref[...]
    w_emb = vecs[0]
    b_emb = vecs[1]
    bn1 = vecs[2]
    bn2 = vecs[3]
    b_out = vecs[4]
    mats = mats_ref[...]
    wn1a = mats[0]
    wn1b = mats[1]
    wn2 = mats[2]
    w_out = mats[3]
    h0 = x * w_emb[None, :] + b_emb[None, :]
    pre = (jnp.dot(h0, wn1a, preferred_element_type=jnp.float32)
           + jnp.dot(agg, wn1b, preferred_element_type=jnp.float32)
           + bn1[None, :])
    t = pre * jax.nn.sigmoid(pre)
    h = h0 + jnp.dot(t, wn2, preferred_element_type=jnp.float32) + bn2[None, :]
    h3_ref[...] = (jnp.dot(h, w_out, preferred_element_type=jnp.float32)
                   + b_out[None, :])


def _node_stage(x2d, agg0, agg1, vecs, mats, nblk=1024):
    n_pad = x2d.shape[0]
    grid = (n_pad // nblk,)
    return pl.pallas_call(
        _node_body,
        grid=grid,
        in_specs=[
            pl.BlockSpec((nblk, 1), lambda i: (i, 0)),
            pl.BlockSpec((nblk, H), lambda i: (i, 0)),
            pl.BlockSpec((nblk, H), lambda i: (i, 0)),
            pl.BlockSpec((8, H), lambda i: (0, 0)),
            pl.BlockSpec((4, H, H), lambda i: (0, 0, 0)),
        ],
        out_specs=pl.BlockSpec((nblk, H), lambda i: (i, 0)),
        out_shape=jax.ShapeDtypeStruct((n_pad, H), jnp.float32),
    )(x2d, agg0, agg1, vecs, mats)


def _z1_body(n_valid, h3_ref, nbr0_ref, nbr1_ref, wg1_ref, bg1_ref, z1_ref,
             sums_ref, acc_ref):
    i = pl.program_id(0)
    nblk = h3_ref.shape[0]
    s = h3_ref[...] + nbr0_ref[...] + nbr1_ref[...]
    z1 = jnp.dot(s, wg1_ref[...], preferred_element_type=jnp.float32) \
        + bg1_ref[...][0][None, :]
    z1_ref[...] = z1
    row = i * nblk + lax.broadcasted_iota(jnp.int32, (nblk, 1), 0)
    zm = jnp.where(row < n_valid, z1, 0.0)

    @pl.when(i == 0)
    def _():
        acc_ref[...] = jnp.zeros_like(acc_ref)

    acc_ref[0:1, :] += jnp.sum(zm, axis=0, keepdims=True)
    acc_ref[1:2, :] += jnp.sum(zm * zm, axis=0, keepdims=True)

    @pl.when(i == pl.num_programs(0) - 1)
    def _():
        sums_ref[...] = acc_ref[...]


def _z1_stage(h3, nbr0, nbr1, wg1, bg1_2d, n_valid, nblk=1024):
    n_pad = h3.shape[0]
    grid = (n_pad // nblk,)
    return pl.pallas_call(
        functools.partial(_z1_body, n_valid),
        grid=grid,
        in_specs=[
            pl.BlockSpec((nblk, H), lambda i: (i, 0)),
            pl.BlockSpec((nblk, H), lambda i: (i, 0)),
            pl.BlockSpec((nblk, H), lambda i: (i, 0)),
            pl.BlockSpec((H, H), lambda i: (0, 0)),
            pl.BlockSpec((1, H), lambda i: (0, 0)),
        ],
        out_specs=[
            pl.BlockSpec((nblk, H), lambda i: (i, 0)),
            pl.BlockSpec((8, H), lambda i: (0, 0)),
        ],
        out_shape=[
            jax.ShapeDtypeStruct((n_pad, H), jnp.float32),
            jax.ShapeDtypeStruct((8, H), jnp.float32),
        ],
        scratch_shapes=[pltpu.VMEM((8, H), jnp.float32)],
    )(h3, nbr0, nbr1, wg1, bg1_2d)


def _final_body(n_valid, g, z1_ref, batch_ref, sums_ref, wg2_ref, wf1_ref,
                wf2_ref, vecs_ref, out_ref, mean_ref, max_ref, cnt_ref):
    i = pl.program_id(0)
    nblk = z1_ref.shape[0]
    vecs = vecs_ref[...]
    gamma = vecs[0]
    beta = vecs[1]
    bg2 = vecs[2]

    @pl.when(i == 0)
    def _():
        mean_ref[...] = jnp.zeros_like(mean_ref)
        max_ref[...] = jnp.full_like(max_ref, -3.0e38)
        cnt_ref[...] = jnp.zeros_like(cnt_ref)

    n_f = jnp.float32(n_valid)
    sums = sums_ref[...]
    mu = sums[0]
    mu = mu / n_f
    var = sums[1] / n_f - mu * mu
    inv = lax.rsqrt(var + 1e-5)
    zn = (z1_ref[...] - mu[None, :]) * (inv * gamma)[None, :] + beta[None, :]
    zn = jnp.maximum(zn, 0.0)
    z = jnp.dot(zn, wg2_ref[...], preferred_element_type=jnp.float32) \
        + bg2[None, :]

    b = batch_ref[0]
    oht = (b == lax.broadcasted_iota(jnp.int32, (g, 1), 0)).astype(jnp.float32)
    mean_ref[...] += lax.dot_general(
        oht, z, (((1,), (0,)), ((), ())),
        preferred_element_type=jnp.float32)
    cnt_ref[...] += jnp.sum(oht, axis=1, keepdims=True)
    masked = jnp.where(oht[:, :, None] > 0, z[None, :, :], -3.0e38)
    max_ref[...] = jnp.maximum(max_ref[...], jnp.max(masked, axis=1))

    @pl.when(i == pl.num_programs(0) - 1)
    def _():
        cnt = cnt_ref[...]
        meanp = mean_ref[...] / jnp.maximum(cnt, 1.0)
        maxp = jnp.where(cnt > 0, max_ref[...], 0.0)
        bf1 = vecs[3]
        bf2 = vecs[4]
        feat = jnp.concatenate(
            [meanp, maxp, cnt,
             jnp.zeros((g, 40 - 2 * H - 1), jnp.float32)], axis=1)
        o = jnp.dot(feat, wf1_ref[...], preferred_element_type=jnp.float32) \
            + bf1[None, :]
        o = jnp.where(o > 0, o, 0.1 * (jnp.exp(o) - 1.0))
        o = jnp.dot(o, wf2_ref[...], preferred_element_type=jnp.float32) \
            + bf2[None, :]
        cmask = lax.broadcasted_iota(jnp.int32, (1, H), 1) < 10
        o = jnp.where(cmask, o, -3.0e38)
        o = o - jnp.max(o, axis=1, keepdims=True)
        e = jnp.exp(o)
        sm = e / jnp.sum(e, axis=1, keepdims=True)
        out_ref[...] = sm


def _final_stage(z1, batch3d, sums, wg2, wf1_pad, wf2_pad, vecs, n_valid, g=64,
                 nblk=512):
    n_pad = z1.shape[0]
    grid = (n_pad // nblk,)
    return pl.pallas_call(
        functools.partial(_final_body, n_valid, g),
        grid=grid,
        in_specs=[
            pl.BlockSpec((nblk, H), lambda i: (i, 0)),
            pl.BlockSpec((1, 1, nblk), lambda i: (i, 0, 0)),
            pl.BlockSpec((8, H), lambda i: (0, 0)),
            pl.BlockSpec((H, H), lambda i: (0, 0)),
            pl.BlockSpec((40, H), lambda i: (0, 0)),
            pl.BlockSpec((H, H), lambda i: (0, 0)),
            pl.BlockSpec((8, H), lambda i: (0, 0)),
        ],
        out_specs=pl.BlockSpec((g, H), lambda i: (0, 0)),
        out_shape=jax.ShapeDtypeStruct((g, H), jnp.float32),
        scratch_shapes=[
            pltpu.VMEM((g, H), jnp.float32),
            pltpu.VMEM((g, H), jnp.float32),
            pltpu.VMEM((g, 1), jnp.float32),
        ],
    )(z1, batch3d, sums, wg2, wf1_pad, wf2_pad, vecs)


def kernel(x, pos, edge_index, batch, W_emb_in, b_emb_in, We1, be1, We2, be2,
           Wc1, bc1, Wc2, Wn1, bn1, Wn2, bn2, W_emb_out, b_emb_out, Wg1, bg1,
           gamma, beta, Wg2, bg2, Wf1, bf1, Wf2, bf2):
    n = x.shape[0]
    e = edge_index.shape[1]
    g = 64
    n_pad = ((n + 1 + 1023) // 1024) * 1024
    e_pad = ((e + 4095) // 4096) * 4096

    row = edge_index[0]
    col = edge_index[1]
    pad_e = e_pad - e
    row_p = jnp.concatenate([row, jnp.full((pad_e,), n, jnp.int32)])
    col_p = jnp.concatenate([col, jnp.full((pad_e,), n, jnp.int32)])

    rec = jnp.concatenate(
        [x, pos, jnp.zeros((n, 4), jnp.float32)], axis=1)
    rec = jnp.concatenate(
        [rec, jnp.zeros((n_pad - n, 8), jnp.float32)], axis=0)

    w_emb = W_emb_in[0]
    aA = w_emb @ We1[:H]
    aB = w_emb @ We1[H:2 * H]
    aR = We1[2 * H]
    c0 = b_emb_in @ We1[:H] + b_emb_in @ We1[H:2 * H] + be1
    edge_vecs = jnp.stack(
        [aA, aB, aR, c0, be2, jnp.zeros_like(aA), jnp.zeros_like(aA),
         jnp.zeros_like(aA)])

    rrow = jnp.take(rec, row_p, axis=0)
    rcol = jnp.take(rec, col_p, axis=0)

    m = _edge_mlp(rrow, rcol, edge_vecs, We2)

    agg = jax.ops.segment_sum(m, row_p, num_segments=n_pad)
    agg0 = agg
    agg1 = jnp.zeros_like(agg)

    x2d = jnp.concatenate([x, jnp.zeros((n_pad - n, 1), jnp.float32)], axis=0)
    node_vecs = jnp.stack(
        [w_emb, b_emb_in, bn1, bn2, b_emb_out, jnp.zeros_like(bn1),
         jnp.zeros_like(bn1), jnp.zeros_like(bn1)])
    node_mats = jnp.stack([Wn1[:H], Wn1[H:], Wn2, W_emb_out])
    h3 = _node_stage(x2d, agg0, agg1, node_vecs, node_mats)

    nbr = jax.ops.segment_sum(jnp.take(h3, col_p, axis=0), row_p,
                              num_segments=n_pad)
    nbr0 = nbr
    nbr1 = jnp.zeros_like(nbr)

    z1, sums = _z1_stage(h3, nbr0, nbr1, Wg1, bg1.reshape(1, H), n)

    batch_p = jnp.concatenate(
        [batch.astype(jnp.int32), jnp.full((n_pad - n,), g, jnp.int32)])
    final_vecs = jnp.stack([gamma, beta, bg2, bf1, jnp.zeros_like(bf1),
                            jnp.zeros_like(bf1), jnp.zeros_like(bf1),
                            jnp.zeros_like(bf1)])
    wf1_pad = jnp.concatenate([Wf1, jnp.zeros((40 - 33, H), jnp.float32)])
    wf2_pad = jnp.concatenate([Wf2, jnp.zeros((H, H - 10), jnp.float32)],
                              axis=1)
    out = _final_stage(z1, batch_p.reshape(-1, 1, 512), sums, Wg2, wf1_pad,
                       wf2_pad, final_vecs, n)
    return out[:, :10]


# speedup vs baseline: 25.3925x; 19.4203x over previous
import functools

import jax
import jax.numpy as jnp
from jax import lax
from jax.experimental import pallas as pl
from jax.experimental.pallas import tpu as pltpu
from jax.experimental.pallas import tpu_sc as plsc

H = 16


_SC_K = 1024


def _sc_gather_body(chunks, n_pad, width, table_hbm, idx_hbm, out_hbm,
                    idx_v, rows_v, tab):
    c = lax.axis_index("c")
    s = lax.axis_index("s")
    tile = c * 16 + s
    base = tile * (chunks * _SC_K)
    stripe = n_pad // 16
    pltpu.sync_copy(table_hbm.at[pl.ds(s * stripe, stripe)],
                    tab.at[pl.ds(s * stripe, stripe)])
    plsc.subcore_barrier()

    def chunk(j, carry):
        off = pl.multiple_of(base + j * _SC_K, _SC_K)
        pltpu.sync_copy(idx_hbm.at[pl.ds(off, _SC_K)], idx_v)
        pltpu.sync_copy(tab.at[idx_v], rows_v)
        pltpu.sync_copy(rows_v, out_hbm.at[pl.ds(off, _SC_K)])
        return carry

    lax.fori_loop(0, chunks, chunk, 0)


def _sc_gather(table, idx, chunks):
    n_pad, width = table.shape
    mesh = plsc.VectorSubcoreMesh(core_axis_name="c", subcore_axis_name="s")
    f = pl.kernel(
        functools.partial(_sc_gather_body, chunks, n_pad, width),
        out_type=jax.ShapeDtypeStruct((idx.shape[0], width), jnp.float32),
        mesh=mesh,
        scratch_types=[
            pltpu.VMEM((_SC_K,), jnp.int32),
            pltpu.VMEM((_SC_K, width), jnp.float32),
            pltpu.VMEM_SHARED((n_pad, width), jnp.float32),
        ],
        compiler_params=pltpu.CompilerParams(use_tc_tiling_on_sc=False),
    )
    return f(table, idx)


def _sc_scatadd_body(chunks, n_pad, vals_hbm, row_hbm, zeros_hbm, out_hbm,
                     row_v, rows_v, acc):
    c = lax.axis_index("c")
    s = lax.axis_index("s")
    tile = c * 16 + s
    base = tile * (chunks * _SC_K)
    stripe = n_pad // 16
    pltpu.sync_copy(zeros_hbm.at[pl.ds(s * stripe, stripe)],
                    acc.at[pl.ds(s * stripe, stripe)])
    plsc.subcore_barrier()

    def chunk(j, carry):
        off = pl.multiple_of(base + j * _SC_K, _SC_K)
        pltpu.sync_copy(row_hbm.at[pl.ds(off, _SC_K)], row_v)
        pltpu.sync_copy(vals_hbm.at[pl.ds(off, _SC_K)], rows_v)
        pltpu.sync_copy(rows_v, acc.at[row_v], add=True)
        return carry

    lax.fori_loop(0, chunks, chunk, 0)
    plsc.subcore_barrier()
    out_off = pl.multiple_of(c * n_pad + s * stripe, 8)
    pltpu.sync_copy(acc.at[pl.ds(s * stripe, stripe)],
                    out_hbm.at[pl.ds(out_off, stripe)])


def _sc_scatadd(vals, row_p, zeros, chunks):
    n_pad = zeros.shape[0]
    mesh = plsc.VectorSubcoreMesh(core_axis_name="c", subcore_axis_name="s")
    f = pl.kernel(
        functools.partial(_sc_scatadd_body, chunks, n_pad),
        out_type=jax.ShapeDtypeStruct((2 * n_pad, H), jnp.float32),
        mesh=mesh,
        scratch_types=[
            pltpu.VMEM((_SC_K,), jnp.int32),
            pltpu.VMEM((_SC_K, H), jnp.float32),
            pltpu.VMEM_SHARED((n_pad, H), jnp.float32),
        ],
        compiler_params=pltpu.CompilerParams(use_tc_tiling_on_sc=False),
    )
    return f(vals, row_p, zeros)


def _edge_mlp_body(rr_ref, rc_ref, mats_ref, vecs_ref, m_ref):
    rr = rr_ref[...]
    rc = rc_ref[...]
    mats = mats_ref[...]
    vecs = vecs_ref[...]
    d = rr - rc
    d2 = d * d * vecs[0][None, :]
    radial_b = jnp.dot(d2, mats[2], preferred_element_type=jnp.float32)
    pre = (jnp.dot(rr, mats[0], preferred_element_type=jnp.float32)
           + jnp.dot(rc, mats[1], preferred_element_type=jnp.float32)
           + radial_b * vecs[1][None, :] + vecs[2][None, :])
    u = pre * jax.nn.sigmoid(pre)
    v = jnp.dot(u, mats[3], preferred_element_type=jnp.float32) \
        + vecs[3][None, :]
    m_ref[...] = v * jax.nn.sigmoid(v)


def _edge_mlp(rrow128, rcol128, mats, vecs, eblk=2048):
    rows = rrow128.shape[0]
    grid = (rows // eblk,)
    return pl.pallas_call(
        _edge_mlp_body,
        grid=grid,
        in_specs=[
            pl.BlockSpec((eblk, 128), lambda i: (i, 0)),
            pl.BlockSpec((eblk, 128), lambda i: (i, 0)),
            pl.BlockSpec((4, 128, 128), lambda i: (0, 0, 0)),
            pl.BlockSpec((8, 128), lambda i: (0, 0)),
        ],
        out_specs=pl.BlockSpec((eblk, 128), lambda i: (i, 0)),
        out_shape=jax.ShapeDtypeStruct((rows, 128), jnp.float32),
    )(rrow128, rcol128, mats, vecs)


def _node_body(x_ref, agg0_ref, agg1_ref, vecs_ref, mats_ref, h3_ref):
    x = x_ref[...]
    agg = agg0_ref[...] + agg1_ref[...]
    vecs = vecs_ref[...]
    w_emb = vecs[0]
    b_emb = vecs[1]
    bn1 = vecs[2]
    bn2 = vecs[3]
    b_out = vecs[4]
    mats = mats_ref[...]
    wn1a = mats[0]
    wn1b = mats[1]
    wn2 = mats[2]
    w_out = mats[3]
    h0 = x * w_emb[None, :] + b_emb[None, :]
    pre = (jnp.dot(h0, wn1a, preferred_element_type=jnp.float32)
           + jnp.dot(agg, wn1b, preferred_element_type=jnp.float32)
           + bn1[None, :])
    t = pre * jax.nn.sigmoid(pre)
    h = h0 + jnp.dot(t, wn2, preferred_element_type=jnp.float32) + bn2[None, :]
    h3_ref[...] = (jnp.dot(h, w_out, preferred_element_type=jnp.float32)
                   + b_out[None, :])


def _node_stage(x2d, agg0, agg1, vecs, mats, nblk=1024):
    n_pad = x2d.shape[0]
    grid = (n_pad // nblk,)
    return pl.pallas_call(
        _node_body,
        grid=grid,
        in_specs=[
            pl.BlockSpec((nblk, 1), lambda i: (i, 0)),
            pl.BlockSpec((nblk, H), lambda i: (i, 0)),
            pl.BlockSpec((nblk, H), lambda i: (i, 0)),
            pl.BlockSpec((8, H), lambda i: (0, 0)),
            pl.BlockSpec((4, H, H), lambda i: (0, 0, 0)),
        ],
        out_specs=pl.BlockSpec((nblk, H), lambda i: (i, 0)),
        out_shape=jax.ShapeDtypeStruct((n_pad, H), jnp.float32),
    )(x2d, agg0, agg1, vecs, mats)


def _z1_body(n_valid, h3_ref, nbr0_ref, nbr1_ref, wg1_ref, bg1_ref, z1_ref,
             sums_ref, acc_ref):
    i = pl.program_id(0)
    nblk = h3_ref.shape[0]
    s = h3_ref[...] + nbr0_ref[...] + nbr1_ref[...]
    z1 = jnp.dot(s, wg1_ref[...], preferred_element_type=jnp.float32) \
        + bg1_ref[...][0][None, :]
    z1_ref[...] = z1
    row = i * nblk + lax.broadcasted_iota(jnp.int32, (nblk, 1), 0)
    zm = jnp.where(row < n_valid, z1, 0.0)

    @pl.when(i == 0)
    def _():
        acc_ref[...] = jnp.zeros_like(acc_ref)

    acc_ref[0:1, :] += jnp.sum(zm, axis=0, keepdims=True)
    acc_ref[1:2, :] += jnp.sum(zm * zm, axis=0, keepdims=True)

    @pl.when(i == pl.num_programs(0) - 1)
    def _():
        sums_ref[...] = acc_ref[...]


def _z1_stage(h3, nbr0, nbr1, wg1, bg1_2d, n_valid, nblk=1024):
    n_pad = h3.shape[0]
    grid = (n_pad // nblk,)
    return pl.pallas_call(
        functools.partial(_z1_body, n_valid),
        grid=grid,
        in_specs=[
            pl.BlockSpec((nblk, H), lambda i: (i, 0)),
            pl.BlockSpec((nblk, H), lambda i: (i, 0)),
            pl.BlockSpec((nblk, H), lambda i: (i, 0)),
            pl.BlockSpec((H, H), lambda i: (0, 0)),
            pl.BlockSpec((1, H), lambda i: (0, 0)),
        ],
        out_specs=[
            pl.BlockSpec((nblk, H), lambda i: (i, 0)),
            pl.BlockSpec((8, H), lambda i: (0, 0)),
        ],
        out_shape=[
            jax.ShapeDtypeStruct((n_pad, H), jnp.float32),
            jax.ShapeDtypeStruct((8, H), jnp.float32),
        ],
        scratch_shapes=[pltpu.VMEM((8, H), jnp.float32)],
    )(h3, nbr0, nbr1, wg1, bg1_2d)


def _final_body(n_valid, g, z1_ref, batch_ref, sums_ref, wg2_ref, wf1_ref,
                wf2_ref, vecs_ref, out_ref, mean_ref, max_ref, cnt_ref):
    i = pl.program_id(0)
    nblk = z1_ref.shape[0]
    vecs = vecs_ref[...]
    gamma = vecs[0]
    beta = vecs[1]
    bg2 = vecs[2]

    @pl.when(i == 0)
    def _():
        mean_ref[...] = jnp.zeros_like(mean_ref)
        max_ref[...] = jnp.full_like(max_ref, -3.0e38)
        cnt_ref[...] = jnp.zeros_like(cnt_ref)

    n_f = jnp.float32(n_valid)
    sums = sums_ref[...]
    mu = sums[0]
    mu = mu / n_f
    var = sums[1] / n_f - mu * mu
    inv = lax.rsqrt(var + 1e-5)
    zn = (z1_ref[...] - mu[None, :]) * (inv * gamma)[None, :] + beta[None, :]
    zn = jnp.maximum(zn, 0.0)
    z = jnp.dot(zn, wg2_ref[...], preferred_element_type=jnp.float32) \
        + bg2[None, :]

    b = batch_ref[0]
    oht = (b == lax.broadcasted_iota(jnp.int32, (g, 1), 0)).astype(jnp.float32)
    mean_ref[...] += lax.dot_general(
        oht, z, (((1,), (0,)), ((), ())),
        preferred_element_type=jnp.float32)
    cnt_ref[...] += jnp.sum(oht, axis=1, keepdims=True)
    masked = jnp.where(oht[:, :, None] > 0, z[None, :, :], -3.0e38)
    max_ref[...] = jnp.maximum(max_ref[...], jnp.max(masked, axis=1))

    @pl.when(i == pl.num_programs(0) - 1)
    def _():
        cnt = cnt_ref[...]
        meanp = mean_ref[...] / jnp.maximum(cnt, 1.0)
        maxp = jnp.where(cnt > 0, max_ref[...], 0.0)
        bf1 = vecs[3]
        bf2 = vecs[4]
        feat = jnp.concatenate(
            [meanp, maxp, cnt,
             jnp.zeros((g, 40 - 2 * H - 1), jnp.float32)], axis=1)
        o = jnp.dot(feat, wf1_ref[...], preferred_element_type=jnp.float32) \
            + bf1[None, :]
        o = jnp.where(o > 0, o, 0.1 * (jnp.exp(o) - 1.0))
        o = jnp.dot(o, wf2_ref[...], preferred_element_type=jnp.float32) \
            + bf2[None, :]
        cmask = lax.broadcasted_iota(jnp.int32, (1, H), 1) < 10
        o = jnp.where(cmask, o, -3.0e38)
        o = o - jnp.max(o, axis=1, keepdims=True)
        e = jnp.exp(o)
        sm = e / jnp.sum(e, axis=1, keepdims=True)
        out_ref[...] = sm


def _final_stage(z1, batch3d, sums, wg2, wf1_pad, wf2_pad, vecs, n_valid, g=64,
                 nblk=512):
    n_pad = z1.shape[0]
    grid = (n_pad // nblk,)
    return pl.pallas_call(
        functools.partial(_final_body, n_valid, g),
        grid=grid,
        in_specs=[
            pl.BlockSpec((nblk, H), lambda i: (i, 0)),
            pl.BlockSpec((1, 1, nblk), lambda i: (i, 0, 0)),
            pl.BlockSpec((8, H), lambda i: (0, 0)),
            pl.BlockSpec((H, H), lambda i: (0, 0)),
            pl.BlockSpec((40, H), lambda i: (0, 0)),
            pl.BlockSpec((H, H), lambda i: (0, 0)),
            pl.BlockSpec((8, H), lambda i: (0, 0)),
        ],
        out_specs=pl.BlockSpec((g, H), lambda i: (0, 0)),
        out_shape=jax.ShapeDtypeStruct((g, H), jnp.float32),
        scratch_shapes=[
            pltpu.VMEM((g, H), jnp.float32),
            pltpu.VMEM((g, H), jnp.float32),
            pltpu.VMEM((g, 1), jnp.float32),
        ],
    )(z1, batch3d, sums, wg2, wf1_pad, wf2_pad, vecs)


def kernel(x, pos, edge_index, batch, W_emb_in, b_emb_in, We1, be1, We2, be2,
           Wc1, bc1, Wc2, Wn1, bn1, Wn2, bn2, W_emb_out, b_emb_out, Wg1, bg1,
           gamma, beta, Wg2, bg2, Wf1, bf1, Wf2, bf2):
    n = x.shape[0]
    e = edge_index.shape[1]
    g = 64
    n_pad = ((n + 1 + 1023) // 1024) * 1024
    sc_chunks = -(-e // (32 * _SC_K))
    e_pad = sc_chunks * 32 * _SC_K

    row = edge_index[0]
    col = edge_index[1]
    pad_e = e_pad - e
    row_p = jnp.concatenate([row.astype(jnp.int32),
                             jnp.full((pad_e,), n, jnp.int32)])
    col_p = jnp.concatenate([col.astype(jnp.int32),
                             jnp.full((pad_e,), n, jnp.int32)])
    zeros_nh = jnp.zeros((n_pad, H), jnp.float32)

    rec = jnp.concatenate(
        [x, pos, jnp.zeros((n, 12), jnp.float32)], axis=1)
    rec = jnp.concatenate(
        [rec, jnp.zeros((n_pad - n, 16), jnp.float32)], axis=0)

    w_emb = W_emb_in[0]
    aA = w_emb @ We1[:H]
    aB = w_emb @ We1[H:2 * H]
    aR = We1[2 * H]
    c0 = b_emb_in @ We1[:H] + b_emb_in @ We1[H:2 * H] + be1

    eye8 = jnp.eye(8, dtype=jnp.float32)
    A16 = jnp.zeros((16, H), jnp.float32).at[0].set(aA)
    B16 = jnp.zeros((16, H), jnp.float32).at[0].set(aB)
    S16 = jnp.zeros((16, 16), jnp.float32).at[1:4].set(1.0)
    W2_16 = jnp.concatenate([We2, jnp.zeros((16 - H, H), jnp.float32)])
    mats = jnp.stack([jnp.kron(eye8, A16), jnp.kron(eye8, B16),
                      jnp.kron(eye8, S16), jnp.kron(eye8, W2_16)])
    dmask16 = jnp.zeros((16,), jnp.float32).at[1:4].set(1.0)
    tile8 = lambda v: jnp.tile(v, 8)
    edge_vecs = jnp.stack(
        [tile8(dmask16), tile8(aR), tile8(c0), tile8(be2),
         jnp.zeros((128,), jnp.float32), jnp.zeros((128,), jnp.float32),
         jnp.zeros((128,), jnp.float32), jnp.zeros((128,), jnp.float32)])

    rrow = _sc_gather(rec, row_p, sc_chunks)
    rcol = _sc_gather(rec, col_p, sc_chunks)

    m128 = _edge_mlp(rrow.reshape(-1, 128), rcol.reshape(-1, 128),
                     mats, edge_vecs)
    m = m128.reshape(e_pad, H)

    agg_pair = _sc_scatadd(m, row_p, zeros_nh, sc_chunks)
    agg0 = agg_pair[:n_pad]
    agg1 = agg_pair[n_pad:]

    x2d = jnp.concatenate([x, jnp.zeros((n_pad - n, 1), jnp.float32)], axis=0)
    node_vecs = jnp.stack(
        [w_emb, b_emb_in, bn1, bn2, b_emb_out, jnp.zeros_like(bn1),
         jnp.zeros_like(bn1), jnp.zeros_like(bn1)])
    node_mats = jnp.stack([Wn1[:H], Wn1[H:], Wn2, W_emb_out])
    h3 = _node_stage(x2d, agg0, agg1, node_vecs, node_mats)

    gathered = _sc_gather(h3, col_p, sc_chunks)
    nbr_pair = _sc_scatadd(gathered, row_p, zeros_nh, sc_chunks)
    nbr0 = nbr_pair[:n_pad]
    nbr1 = nbr_pair[n_pad:]

    z1, sums = _z1_stage(h3, nbr0, nbr1, Wg1, bg1.reshape(1, H), n)

    batch_p = jnp.concatenate(
        [batch.astype(jnp.int32), jnp.full((n_pad - n,), g, jnp.int32)])
    final_vecs = jnp.stack([gamma, beta, bg2, bf1, jnp.zeros_like(bf1),
                            jnp.zeros_like(bf1), jnp.zeros_like(bf1),
                            jnp.zeros_like(bf1)])
    wf1_pad = jnp.concatenate([Wf1, jnp.zeros((40 - 33, H), jnp.float32)])
    wf2_pad = jnp.concatenate([Wf2, jnp.zeros((H, H - 10), jnp.float32)],
                              axis=1)
    out = _final_stage(z1, batch_p.reshape(-1, 1, 512), sums, Wg2, wf1_pad,
                       wf2_pad, final_vecs, n)
    return out[:, :10]
